# baseline (device time: 24037 ns/iter reference)
import jax
import jax.numpy as jnp
from jax import lax
from jax.experimental import pallas as pl
from jax.experimental.pallas import tpu as pltpu

T = 512
D = 1024
V_SHARD = 8192
HALF_COLS = V_SHARD // 2
CHUNK = 1024
N_CH = HALF_COLS // CHUNK


def _body(x_ref, w_hbm, lab_ref, out_ref, wbuf, stats1, comm1, stats2, comm2,
          copy_sems, send1, recv1, send2, recv2):
    my_x = lax.axis_index("x")
    my_y = lax.axis_index("y")
    col0 = my_y * HALF_COLS

    copies = []
    for c in range(N_CH):
        cp = pltpu.make_async_copy(
            w_hbm.at[:, pl.ds(col0 + c * CHUNK, CHUNK)],
            wbuf.at[c],
            copy_sems.at[c],
        )
        cp.start()
        copies.append(cp)

    xv = x_ref[:, :]
    local_lab = lab_ref[:, :] - my_x * V_SHARD

    s_acc = jnp.zeros((T, 1), jnp.float32)
    lab_acc = jnp.zeros((T, 1), jnp.float32)
    for c in range(N_CH):
        copies[c].wait()
        logits = jnp.dot(
            xv, wbuf[c],
            preferred_element_type=jnp.float32,
            precision=lax.Precision.DEFAULT,
        )
        col = lax.broadcasted_iota(jnp.int32, (T, CHUNK), 1) + (col0 + c * CHUNK)
        s_acc = s_acc + jnp.sum(jnp.exp(logits), axis=1, keepdims=True)
        lab_acc = lab_acc + jnp.sum(
            jnp.where(col == local_lab, logits, 0.0), axis=1, keepdims=True
        )

    x_peer = (1 - my_x, my_y)
    y_peer = (my_x, 1 - my_y)

    barrier_sem = pltpu.get_barrier_semaphore()
    for peer in (x_peer, y_peer):
        pl.semaphore_signal(
            barrier_sem, inc=1, device_id=peer,
            device_id_type=pl.DeviceIdType.MESH,
        )
    pl.semaphore_wait(barrier_sem, 2)

    stats1[:, 0:1] = s_acc
    stats1[:, 1:2] = lab_acc
    rdma1 = pltpu.make_async_remote_copy(
        src_ref=stats1, dst_ref=comm1, send_sem=send1, recv_sem=recv1,
        device_id=x_peer, device_id_type=pl.DeviceIdType.MESH,
    )
    rdma1.start()
    rdma1.wait()
    s2 = s_acc + comm1[:, 0:1]
    lab2 = lab_acc + comm1[:, 1:2]

    stats2[:, 0:1] = s2
    stats2[:, 1:2] = lab2
    rdma2 = pltpu.make_async_remote_copy(
        src_ref=stats2, dst_ref=comm2, send_sem=send2, recv_sem=recv2,
        device_id=y_peer, device_id_type=pl.DeviceIdType.MESH,
    )
    rdma2.start()
    rdma2.wait()
    s_tot = s2 + comm2[:, 0:1]
    lab_tot = lab2 + comm2[:, 1:2]

    nll = jnp.log(s_tot) - lab_tot
    out_ref[:] = nll[:, 0]


def kernel(x, W, labels):
    labels2d = labels.reshape(T, 1)
    return pl.pallas_call(
        _body,
        in_specs=[
            pl.BlockSpec(memory_space=pltpu.MemorySpace.VMEM),
            pl.BlockSpec(memory_space=pl.ANY),
            pl.BlockSpec(memory_space=pltpu.MemorySpace.VMEM),
        ],
        out_specs=pl.BlockSpec(memory_space=pltpu.MemorySpace.VMEM),
        out_shape=jax.ShapeDtypeStruct((T,), jnp.float32),
        scratch_shapes=[
            pltpu.VMEM((N_CH, D, CHUNK), jnp.float32),
            pltpu.VMEM((T, 2), jnp.float32),
            pltpu.VMEM((T, 2), jnp.float32),
            pltpu.VMEM((T, 2), jnp.float32),
            pltpu.VMEM((T, 2), jnp.float32),
            pltpu.SemaphoreType.DMA((N_CH,)),
            pltpu.SemaphoreType.DMA,
            pltpu.SemaphoreType.DMA,
            pltpu.SemaphoreType.DMA,
            pltpu.SemaphoreType.DMA,
        ],
        compiler_params=pltpu.CompilerParams(collective_id=0),
    )(x, W, labels2d)


# device time: 22672 ns/iter; 1.0602x vs baseline; 1.0602x over previous
import jax
import jax.numpy as jnp
from jax import lax
from jax.experimental import pallas as pl
from jax.experimental.pallas import tpu as pltpu

T = 512
D = 1024
V_SHARD = 8192
HALF_COLS = V_SHARD // 2
CHUNK = 1024
N_CH = HALF_COLS // CHUNK
SLABS = 4
SLAB_ROWS = D // SLABS


def _body(x_ref, w_hbm, lab_ref, out_ref, wbuf, stats, comm,
          copy_sems, send_sems, recv_sems):
    my_x = lax.axis_index("x")
    my_y = lax.axis_index("y")
    col0 = my_y * HALF_COLS

    peers = (
        (1 - my_x, my_y),
        (my_x, 1 - my_y),
        (1 - my_x, 1 - my_y),
    )

    barrier_sem = pltpu.get_barrier_semaphore()
    for peer in peers:
        pl.semaphore_signal(
            barrier_sem, inc=1, device_id=peer,
            device_id_type=pl.DeviceIdType.MESH,
        )

    copies = []
    for c in range(N_CH):
        for s in range(SLABS):
            k = c * SLABS + s
            cp = pltpu.make_async_copy(
                w_hbm.at[pl.ds(s * SLAB_ROWS, SLAB_ROWS),
                         pl.ds(col0 + c * CHUNK, CHUNK)],
                wbuf.at[c, pl.ds(s * SLAB_ROWS, SLAB_ROWS), :],
                copy_sems.at[k],
            )
            cp.start()
            copies.append(cp)

    xv = x_ref[:, :]
    local_lab = lab_ref[:, :] - my_x * V_SHARD

    s_acc = jnp.zeros((T, 1), jnp.float32)
    lab_acc = jnp.zeros((T, 1), jnp.float32)
    for c in range(N_CH):
        for s in range(SLABS):
            copies[c * SLABS + s].wait()
        logits = jnp.dot(
            xv, wbuf[c],
            preferred_element_type=jnp.float32,
            precision=lax.Precision.DEFAULT,
        )
        col = lax.broadcasted_iota(jnp.int32, (T, CHUNK), 1) + (col0 + c * CHUNK)
        s_acc = s_acc + jnp.sum(jnp.exp(logits), axis=1, keepdims=True)
        lab_acc = lab_acc + jnp.sum(
            jnp.where(col == local_lab, logits, 0.0), axis=1, keepdims=True
        )

    stats[:, 0:1] = s_acc
    stats[:, 1:2] = lab_acc

    pl.semaphore_wait(barrier_sem, 3)

    rdmas = []
    for k, peer in enumerate(peers):
        rdma = pltpu.make_async_remote_copy(
            src_ref=stats,
            dst_ref=comm.at[k],
            send_sem=send_sems.at[k],
            recv_sem=recv_sems.at[k],
            device_id=peer,
            device_id_type=pl.DeviceIdType.MESH,
        )
        rdma.start()
        rdmas.append(rdma)
    for rdma in rdmas:
        rdma.wait()

    s_tot = s_acc + comm[0, :, 0:1] + comm[1, :, 0:1] + comm[2, :, 0:1]
    lab_tot = lab_acc + comm[0, :, 1:2] + comm[1, :, 1:2] + comm[2, :, 1:2]

    nll = jnp.log(s_tot) - lab_tot
    out_ref[:] = nll[:, 0]


def kernel(x, W, labels):
    labels2d = labels.reshape(T, 1)
    return pl.pallas_call(
        _body,
        in_specs=[
            pl.BlockSpec(memory_space=pltpu.MemorySpace.VMEM),
            pl.BlockSpec(memory_space=pl.ANY),
            pl.BlockSpec(memory_space=pltpu.MemorySpace.VMEM),
        ],
        out_specs=pl.BlockSpec(memory_space=pltpu.MemorySpace.VMEM),
        out_shape=jax.ShapeDtypeStruct((T,), jnp.float32),
        scratch_shapes=[
            pltpu.VMEM((N_CH, D, CHUNK), jnp.float32),
            pltpu.VMEM((T, 2), jnp.float32),
            pltpu.VMEM((3, T, 2), jnp.float32),
            pltpu.SemaphoreType.DMA((N_CH * SLABS,)),
            pltpu.SemaphoreType.DMA((3,)),
            pltpu.SemaphoreType.DMA((3,)),
        ],
        compiler_params=pltpu.CompilerParams(collective_id=0),
    )(x, W, labels2d)


# device time: 18318 ns/iter; 1.3122x vs baseline; 1.2377x over previous
import jax
import jax.numpy as jnp
from jax import lax
from jax.experimental import pallas as pl
from jax.experimental.pallas import tpu as pltpu

T = 512
D = 1024
V_SHARD = 8192
HALF_COLS = V_SHARD // 2
CHUNK = 1024
N_CH = HALF_COLS // CHUNK
SLABS = 4
SLAB_ROWS = D // SLABS


def _body(x_ref, w_hbm, lab_ref, out_ref, wbuf, stats, comm,
          copy_sems, send_sems, recv_sems):
    my_x = lax.axis_index("x")
    my_y = lax.axis_index("y")
    col0 = my_y * HALF_COLS

    peers = (
        (1 - my_x, my_y),
        (my_x, 1 - my_y),
        (1 - my_x, 1 - my_y),
    )

    barrier_sem = pltpu.get_barrier_semaphore()
    for peer in peers:
        pl.semaphore_signal(
            barrier_sem, inc=1, device_id=peer,
            device_id_type=pl.DeviceIdType.MESH,
        )

    copies = []
    for c in range(N_CH):
        for s in range(SLABS):
            cp = pltpu.make_async_copy(
                w_hbm.at[pl.ds(s * SLAB_ROWS, SLAB_ROWS),
                         pl.ds(col0 + c * CHUNK, CHUNK)],
                wbuf.at[c, pl.ds(s * SLAB_ROWS, SLAB_ROWS), :],
                copy_sems.at[c * SLABS + s],
            )
            cp.start()
            copies.append(cp)

    xv = x_ref[:, :]
    local_lab = lab_ref[:, :] - my_x * V_SHARD

    s_acc = jnp.zeros((T, 1), jnp.float32)
    lab_acc = jnp.zeros((T, 1), jnp.float32)
    for c in range(N_CH):
        for s in range(SLABS):
            copies[c * SLABS + s].wait()
        logits = jnp.dot(
            xv, wbuf[c],
            preferred_element_type=jnp.float32,
            precision=lax.Precision.DEFAULT,
        )
        col = lax.broadcasted_iota(jnp.int32, (T, CHUNK), 1) + (col0 + c * CHUNK)
        s_acc = s_acc + jnp.sum(jnp.exp(logits), axis=1, keepdims=True)
        lab_acc = lab_acc + jnp.sum(
            jnp.where(col == local_lab, logits, 0.0), axis=1, keepdims=True
        )

    s_row = s_acc[:, 0]
    lab_row = lab_acc[:, 0]
    stats[0, :] = s_row
    stats[1, :] = lab_row

    pl.semaphore_wait(barrier_sem, 3)

    rdmas = []
    for k, peer in enumerate(peers):
        rdma = pltpu.make_async_remote_copy(
            src_ref=stats,
            dst_ref=comm.at[k],
            send_sem=send_sems.at[k],
            recv_sem=recv_sems.at[k],
            device_id=peer,
            device_id_type=pl.DeviceIdType.MESH,
        )
        rdma.start()
        rdmas.append(rdma)
    for rdma in rdmas:
        rdma.wait()

    s_tot = s_row + comm[0, 0, :] + comm[1, 0, :] + comm[2, 0, :]
    lab_tot = lab_row + comm[0, 1, :] + comm[1, 1, :] + comm[2, 1, :]

    out_ref[:] = jnp.log(s_tot) - lab_tot


def kernel(x, W, labels):
    labels2d = labels.reshape(T, 1)
    return pl.pallas_call(
        _body,
        in_specs=[
            pl.BlockSpec(memory_space=pltpu.MemorySpace.VMEM),
            pl.BlockSpec(memory_space=pl.ANY),
            pl.BlockSpec(memory_space=pltpu.MemorySpace.VMEM),
        ],
        out_specs=pl.BlockSpec(memory_space=pltpu.MemorySpace.VMEM),
        out_shape=jax.ShapeDtypeStruct((T,), jnp.float32),
        scratch_shapes=[
            pltpu.VMEM((N_CH, D, CHUNK), jnp.float32),
            pltpu.VMEM((8, T), jnp.float32),
            pltpu.VMEM((3, 8, T), jnp.float32),
            pltpu.SemaphoreType.DMA((N_CH * SLABS,)),
            pltpu.SemaphoreType.DMA((3,)),
            pltpu.SemaphoreType.DMA((3,)),
        ],
        compiler_params=pltpu.CompilerParams(collective_id=0),
    )(x, W, labels2d)
